# baseline (speedup 1.0000x reference)
import functools

import jax
import jax.numpy as jnp
from jax import lax
from jax.experimental import pallas as pl
from jax.experimental.pallas import tpu as pltpu
from jax.experimental.pallas import tpu_sc as plsc

R = 4
P = 6
H = R * P
HP = H // 2
D = 64
B = 32768

NC = 2
NS = 16
NW = NC * NS
L = 16
BPW = B // NW

TC_BLK = 8192


def _round_bf16_bits(x):
    u = lax.bitcast_convert_type(x, jnp.uint32)
    return (u + jnp.uint32(0x7FFF) + ((u >> 16) & jnp.uint32(1))) >> 16


def _logits_tc_kernel(z_ref, w_ref, b_ref, out_ref):
    acc = lax.dot_general(
        w_ref[...], z_ref[...],
        dimension_numbers=(((1,), (1,)), ((), ())),
        preferred_element_type=jnp.float32,
    )
    acc = acc + b_ref[...]
    lo = _round_bf16_bits(acc[0:HP, :])
    hi = _round_bf16_bits(acc[HP:H, :])
    out_ref[...] = lax.bitcast_convert_type(lo | (hi << 16), jnp.int32)


def _compute_logits(z, w_flat, b_flat):
    return pl.pallas_call(
        _logits_tc_kernel,
        grid=(B // TC_BLK,),
        in_specs=[
            pl.BlockSpec((TC_BLK, D), lambda i: (i, 0)),
            pl.BlockSpec((H, D), lambda i: (0, 0)),
            pl.BlockSpec((H, 1), lambda i: (0, 0)),
        ],
        out_specs=pl.BlockSpec((HP, TC_BLK), lambda i: (0, i)),
        out_shape=jax.ShapeDtypeStruct((HP, B), jnp.int32),
    )(z, w_flat, b_flat)


def _route_sc(plane, pair_idx):
    mesh = plsc.VectorSubcoreMesh(core_axis_name="c", subcore_axis_name="s")

    @functools.partial(
        pl.kernel,
        mesh=mesh,
        out_type=tuple(
            jax.ShapeDtypeStruct((B,), jnp.float32) for _ in range(R)
        ),
        scratch_types=[
            pltpu.VMEM((BPW,), jnp.int32),
            pltpu.VMEM((HP, BPW), jnp.int32),
            pltpu.VMEM((R, BPW), jnp.float32),
        ],
        compiler_params=pltpu.CompilerParams(needs_layout_passes=False),
    )
    def route(plane_hbm, pair_hbm, o0, o1, o2, o3, idx_v, plane_v, out_v):
        wid = lax.axis_index("s") * NC + lax.axis_index("c")
        base = wid * BPW
        pltpu.sync_copy(pair_hbm.at[pl.ds(base, BPW)], idx_v)
        pltpu.sync_copy(plane_hbm.at[:, pl.ds(base, BPW)], plane_v)

        himask = jnp.full((L,), -65536, jnp.int32)

        def body(g, _):
            off = g * L
            p16 = idx_v[pl.ds(off, L)]
            cols = off + lax.iota(jnp.int32, L)
            v0 = plsc.load_gather(plane_v, [p16, cols])
            v1 = plsc.load_gather(plane_v, [p16 + P, cols])
            out_v[0, pl.ds(off, L)] = plsc.bitcast(v0 << 16, jnp.float32)
            out_v[1, pl.ds(off, L)] = plsc.bitcast(v1 << 16, jnp.float32)
            out_v[2, pl.ds(off, L)] = plsc.bitcast(v0 & himask, jnp.float32)
            out_v[3, pl.ds(off, L)] = plsc.bitcast(v1 & himask, jnp.float32)
            return 0

        lax.fori_loop(0, BPW // L, body, 0)
        for r, o in enumerate((o0, o1, o2, o3)):
            pltpu.sync_copy(out_v.at[r], o.at[pl.ds(base, BPW)])

    return route(plane, pair_idx)


def kernel(z, pair_idx, W, b):
    w_flat = W.reshape(H, D)
    b_col = b.reshape(H, 1)
    plane = _compute_logits(z, w_flat, b_col)
    return _route_sc(plane, pair_idx.astype(jnp.int32))


# speedup vs baseline: 1.0159x; 1.0159x over previous
import functools

import jax
import jax.numpy as jnp
from jax import lax
from jax.experimental import pallas as pl
from jax.experimental.pallas import tpu as pltpu
from jax.experimental.pallas import tpu_sc as plsc

R = 4
P = 6
H = R * P
HP = H // 2
D = 64
B = 32768

NC = 2
NS = 16
NW = NC * NS
L = 16
BPW = B // NW

TC_BLK = 16384


def _round_bf16_bits(x):
    u = lax.bitcast_convert_type(x, jnp.uint32)
    return (u + jnp.uint32(0x7FFF) + ((u >> 16) & jnp.uint32(1))) >> 16


def _logits_tc_kernel(z_ref, w_ref, b_ref, out_ref):
    acc = lax.dot_general(
        w_ref[...], z_ref[...],
        dimension_numbers=(((1,), (1,)), ((), ())),
        preferred_element_type=jnp.float32,
    )
    acc = acc + b_ref[...]
    lo = _round_bf16_bits(acc[0:HP, :])
    hi = _round_bf16_bits(acc[HP:H, :])
    out_ref[...] = lax.bitcast_convert_type(lo | (hi << 16), jnp.int32)


def _compute_logits(z, w_flat, b_flat):
    return pl.pallas_call(
        _logits_tc_kernel,
        grid=(B // TC_BLK,),
        in_specs=[
            pl.BlockSpec((TC_BLK, D), lambda i: (i, 0)),
            pl.BlockSpec((H, D), lambda i: (0, 0)),
            pl.BlockSpec((H, 1), lambda i: (0, 0)),
        ],
        out_specs=pl.BlockSpec((HP, TC_BLK), lambda i: (0, i)),
        out_shape=jax.ShapeDtypeStruct((HP, B), jnp.int32),
    )(z, w_flat, b_flat)


def _route_sc(plane, pair_idx):
    mesh = plsc.VectorSubcoreMesh(core_axis_name="c", subcore_axis_name="s")

    @functools.partial(
        pl.kernel,
        mesh=mesh,
        out_type=tuple(
            jax.ShapeDtypeStruct((B,), jnp.float32) for _ in range(R)
        ),
        scratch_types=[
            pltpu.VMEM((BPW,), jnp.int32),
            pltpu.VMEM((HP, BPW), jnp.int32),
            pltpu.VMEM((R, BPW), jnp.float32),
            pltpu.SemaphoreType.DMA,
            pltpu.SemaphoreType.DMA,
            pltpu.SemaphoreType.DMA,
        ],
        compiler_params=pltpu.CompilerParams(needs_layout_passes=False),
    )
    def route(plane_hbm, pair_hbm, o0, o1, o2, o3,
              idx_v, plane_v, out_v, sem_p, sem_a, sem_b):
        wid = lax.axis_index("s") * NC + lax.axis_index("c")
        base = wid * BPW
        half = BPW // 2
        c_pair = pltpu.async_copy(pair_hbm.at[pl.ds(base, BPW)], idx_v, sem_p)
        c_a = pltpu.async_copy(
            plane_hbm.at[:, pl.ds(base, half)],
            plane_v.at[:, pl.ds(0, half)], sem_a)
        c_b = pltpu.async_copy(
            plane_hbm.at[:, pl.ds(base + half, half)],
            plane_v.at[:, pl.ds(half, half)], sem_b)

        himask = jnp.full((L,), -65536, jnp.int32)

        def body(g, _):
            off = g * L
            p16 = idx_v[pl.ds(off, L)]
            cols = off + lax.iota(jnp.int32, L)
            v0 = plsc.load_gather(plane_v, [p16, cols])
            v1 = plsc.load_gather(plane_v, [p16 + P, cols])
            out_v[0, pl.ds(off, L)] = plsc.bitcast(v0 << 16, jnp.float32)
            out_v[1, pl.ds(off, L)] = plsc.bitcast(v1 << 16, jnp.float32)
            out_v[2, pl.ds(off, L)] = plsc.bitcast(v0 & himask, jnp.float32)
            out_v[3, pl.ds(off, L)] = plsc.bitcast(v1 & himask, jnp.float32)
            return 0

        c_pair.wait()
        c_a.wait()
        lax.fori_loop(0, half // L, body, 0)
        c_b.wait()
        lax.fori_loop(half // L, BPW // L, body, 0)
        for r, o in enumerate((o0, o1, o2, o3)):
            pltpu.sync_copy(out_v.at[r], o.at[pl.ds(base, BPW)])

    return route(plane, pair_idx)


def kernel(z, pair_idx, W, b):
    w_flat = W.reshape(H, D)
    b_col = b.reshape(H, 1)
    plane = _compute_logits(z, w_flat, b_col)
    return _route_sc(plane, pair_idx.astype(jnp.int32))


# speedup vs baseline: 1.6027x; 1.5777x over previous
import functools

import jax
import jax.numpy as jnp
from jax import lax
from jax.experimental import pallas as pl
from jax.experimental.pallas import tpu as pltpu
from jax.experimental.pallas import tpu_sc as plsc

R = 4
P = 6
H = R * P
HP = H // 2
D = 64
B = 32768

NC = 2
NS = 16
NW = NC * NS
L = 16
BPW = B // NW

TC_BLK = 16384


def _round_bf16_bits(x):
    u = lax.bitcast_convert_type(x, jnp.uint32)
    return (u + jnp.uint32(0x7FFF) + ((u >> 16) & jnp.uint32(1))) >> 16


def _logits_tc_kernel(zt_ref, w_ref, b_ref, out_ref):
    acc = lax.dot_general(
        w_ref[...], zt_ref[...],
        dimension_numbers=(((1,), (0,)), ((), ())),
        preferred_element_type=jnp.float32,
    )
    acc = acc + b_ref[...]
    lo = _round_bf16_bits(acc[0:HP, :])
    hi = _round_bf16_bits(acc[HP:H, :])
    out_ref[...] = lax.bitcast_convert_type(lo | (hi << 16), jnp.int32)


def _compute_logits(zt, w_flat, b_flat):
    return pl.pallas_call(
        _logits_tc_kernel,
        grid=(B // TC_BLK,),
        in_specs=[
            pl.BlockSpec((D, TC_BLK), lambda i: (0, i)),
            pl.BlockSpec((H, D), lambda i: (0, 0)),
            pl.BlockSpec((H, 1), lambda i: (0, 0)),
        ],
        out_specs=pl.BlockSpec((HP, TC_BLK), lambda i: (0, i)),
        out_shape=jax.ShapeDtypeStruct((HP, B), jnp.int32),
    )(zt, w_flat, b_flat)


def _route_sc(plane, pair_idx):
    mesh = plsc.VectorSubcoreMesh(core_axis_name="c", subcore_axis_name="s")

    @functools.partial(
        pl.kernel,
        mesh=mesh,
        out_type=tuple(
            jax.ShapeDtypeStruct((B,), jnp.float32) for _ in range(R)
        ),
        scratch_types=[
            pltpu.VMEM((BPW,), jnp.int32),
            pltpu.VMEM((HP, BPW), jnp.int32),
            pltpu.VMEM((R, BPW), jnp.float32),
            pltpu.SemaphoreType.DMA,
            pltpu.SemaphoreType.DMA,
            pltpu.SemaphoreType.DMA,
        ],
        compiler_params=pltpu.CompilerParams(needs_layout_passes=False),
    )
    def route(plane_hbm, pair_hbm, o0, o1, o2, o3,
              idx_v, plane_v, out_v, sem_p, sem_a, sem_b):
        wid = lax.axis_index("s") * NC + lax.axis_index("c")
        base = wid * BPW
        half = BPW // 2
        c_pair = pltpu.async_copy(pair_hbm.at[pl.ds(base, BPW)], idx_v, sem_p)
        c_a = pltpu.async_copy(
            plane_hbm.at[:, pl.ds(base, half)],
            plane_v.at[:, pl.ds(0, half)], sem_a)
        c_b = pltpu.async_copy(
            plane_hbm.at[:, pl.ds(base + half, half)],
            plane_v.at[:, pl.ds(half, half)], sem_b)

        himask = jnp.full((L,), -65536, jnp.int32)

        def body(g, _):
            off = g * L
            p16 = idx_v[pl.ds(off, L)]
            cols = off + lax.iota(jnp.int32, L)
            v0 = plsc.load_gather(plane_v, [p16, cols])
            v1 = plsc.load_gather(plane_v, [p16 + P, cols])
            out_v[0, pl.ds(off, L)] = plsc.bitcast(v0 << 16, jnp.float32)
            out_v[1, pl.ds(off, L)] = plsc.bitcast(v1 << 16, jnp.float32)
            out_v[2, pl.ds(off, L)] = plsc.bitcast(v0 & himask, jnp.float32)
            out_v[3, pl.ds(off, L)] = plsc.bitcast(v1 & himask, jnp.float32)
            return 0

        c_pair.wait()
        c_a.wait()
        lax.fori_loop(0, half // L, body, 0)
        c_b.wait()
        lax.fori_loop(half // L, BPW // L, body, 0)
        for r, o in enumerate((o0, o1, o2, o3)):
            pltpu.sync_copy(out_v.at[r], o.at[pl.ds(base, BPW)])

    return route(plane, pair_idx)


def kernel(z, pair_idx, W, b):
    w_flat = W.reshape(H, D)
    b_col = b.reshape(H, 1)
    plane = _compute_logits(z.T, w_flat, b_col)
    return _route_sc(plane, pair_idx.astype(jnp.int32))
